# baseline (speedup 1.0000x reference)
import functools

import jax
import jax.numpy as jnp
from jax import lax
from jax.experimental import pallas as pl
from jax.experimental.pallas import tpu as pltpu
from jax.experimental.pallas import tpu_sc as plsc

TOPK = 2
BLK = 256
HT = 4


def _sc_row_gather(npad, n, d):
    info = plsc.get_sparse_core_info()
    nw = info.num_cores * info.num_subcores
    per_w = npad // nw
    c1 = (per_w // 2) & ~7
    c2 = per_w - c1

    mesh = plsc.VectorSubcoreMesh(core_axis_name="c", subcore_axis_name="s")

    @functools.partial(
        pl.kernel, mesh=mesh,
        out_type=jax.ShapeDtypeStruct((npad, d), jnp.float32),
        scratch_types=[
            pltpu.VMEM((c1,), jnp.int32),
            pltpu.VMEM((c2,), jnp.int32),
            pltpu.VMEM((c2, d), jnp.float32),
            pltpu.SemaphoreType.DMA,
        ],
    )
    def gather(table_hbm, idx_hbm, out_hbm, idx_a, idx_b, rows, sem):
        wid = lax.axis_index("s") * info.num_cores + lax.axis_index("c")
        base = wid * per_w
        pltpu.sync_copy(idx_hbm.at[pl.ds(base, c1)], idx_a)
        pltpu.async_copy(table_hbm.at[idx_a], rows.at[pl.ds(0, c1)], sem).wait()
        pltpu.sync_copy(rows.at[pl.ds(0, c1)], out_hbm.at[pl.ds(base, c1)])
        pltpu.sync_copy(idx_hbm.at[pl.ds(base + c1, c2)], idx_b)
        pltpu.async_copy(table_hbm.at[idx_b], rows, sem).wait()
        pltpu.sync_copy(rows, out_hbm.at[pl.ds(base + c1, c2)])

    return gather


def _moe_body(m_ref, x_ref, w1_ref, b1_ref, w2_ref, b2_ref, w3_ref, b3_ref,
              gt_ref, out_ref, w1b, w2b, w3b, hb, *, nb, htile):
    g = pl.program_id(0)
    t = pl.program_id(1)

    @pl.when(g < m_ref[0, nb])
    def _():
        @pl.when(m_ref[1, g] == 1)
        def _():
            w2b[:, pl.ds(t * htile, htile)] = w2_ref[0].astype(jnp.bfloat16)
            w3b[pl.ds(t * htile, htile), :] = w3_ref[0].astype(jnp.bfloat16)

            @pl.when(t == 0)
            def _():
                w1b[...] = w1_ref[0].astype(jnp.bfloat16)

        @pl.when(t == 0)
        def _():
            h = jnp.dot(x_ref[...].astype(jnp.bfloat16), w1b[...],
                        preferred_element_type=jnp.float32) + b1_ref[0]
            h = h * 0.5 * (1.0 + lax.erf(h * 0.7071067811865476))
            hb[...] = h.astype(jnp.bfloat16)

        h2 = jnp.dot(hb[...], w2b[:, pl.ds(t * htile, htile)],
                     preferred_element_type=jnp.float32) + b2_ref[0]
        part = jnp.dot(h2.astype(jnp.bfloat16), w3b[pl.ds(t * htile, htile), :],
                       preferred_element_type=jnp.float32)

        @pl.when(t == 0)
        def _():
            out_ref[...] = (part + b3_ref[0]) * gt_ref[...]

        @pl.when(t != 0)
        def _():
            out_ref[...] += part * gt_ref[...]


def kernel(z_pred, expert_eligibility, W1, b1, W2, b2, W3, b3):
    n, d = z_pred.shape
    e = W1.shape[0]
    h_dim = W1.shape[2]
    htile = h_dim // HT
    np_ = n * TOPK
    nb = np_ // BLK + e - 1
    npad = nb * BLK

    vals, idx = lax.top_k(expert_eligibility, TOPK)
    gates = jax.nn.softmax(vals, axis=-1)
    e_flat = idx.reshape(-1).astype(jnp.int32)
    g_flat = gates.reshape(-1)

    onehot = (e_flat[:, None] == jnp.arange(e, dtype=jnp.int32)[None, :])
    csum = jnp.cumsum(onehot.astype(jnp.int32), axis=0)
    rank = jnp.take_along_axis(csum, e_flat[:, None], axis=1)[:, 0] - 1
    counts = csum[-1]
    nblk_e = (counts + BLK - 1) // BLK
    cum_blocks = jnp.cumsum(nblk_e)
    total_blocks = cum_blocks[-1]
    poffs = jnp.concatenate([jnp.zeros(1, jnp.int32),
                             jnp.cumsum(nblk_e * BLK)])[:e]
    dest = poffs[e_flat] + rank

    bids = jnp.arange(nb, dtype=jnp.int32)
    block_expert = jnp.searchsorted(
        cum_blocks, jnp.minimum(bids, total_blocks - 1), side="right"
    ).astype(jnp.int32)
    changed = jnp.concatenate([jnp.ones(1, jnp.int32),
                               (jnp.diff(block_expert) != 0).astype(jnp.int32)])
    meta = jnp.stack([jnp.concatenate([block_expert, total_blocks[None]]),
                      jnp.concatenate([changed, jnp.zeros(1, jnp.int32)])])

    tok_src = jnp.zeros(npad, jnp.int32).at[dest].set(
        jnp.arange(np_, dtype=jnp.int32) // TOPK)
    gate_col = jnp.zeros((npad,), jnp.float32).at[dest].set(
        g_flat).reshape(npad, 1)
    x_sorted = _sc_row_gather(npad, n, d)(z_pred, tok_src)

    def w2_map(g, t, m):
        return (m[0, g], 0, jnp.where(m[1, g] == 1, t, HT - 1))

    def w3_map(g, t, m):
        return (m[0, g], jnp.where(m[1, g] == 1, t, HT - 1), 0)

    grid_spec = pltpu.PrefetchScalarGridSpec(
        num_scalar_prefetch=1,
        grid=(nb, HT),
        in_specs=[
            pl.BlockSpec((BLK, d), lambda g, t, m: (g, 0)),
            pl.BlockSpec((1, d, h_dim), lambda g, t, m: (m[0, g], 0, 0)),
            pl.BlockSpec((1, 1, h_dim), lambda g, t, m: (m[0, g], 0, 0)),
            pl.BlockSpec((1, h_dim, htile), w2_map),
            pl.BlockSpec((1, 1, htile), lambda g, t, m: (m[0, g], 0, t)),
            pl.BlockSpec((1, htile, d), w3_map),
            pl.BlockSpec((1, 1, d), lambda g, t, m: (m[0, g], 0, 0)),
            pl.BlockSpec((BLK, 1), lambda g, t, m: (g, 0)),
        ],
        out_specs=pl.BlockSpec((BLK, d), lambda g, t, m: (g, 0)),
        scratch_shapes=[
            pltpu.VMEM((d, h_dim), jnp.bfloat16),
            pltpu.VMEM((h_dim, h_dim), jnp.bfloat16),
            pltpu.VMEM((h_dim, d), jnp.bfloat16),
            pltpu.VMEM((BLK, h_dim), jnp.bfloat16),
        ],
    )
    out_rows = pl.pallas_call(
        functools.partial(_moe_body, nb=nb, htile=htile),
        grid_spec=grid_spec,
        out_shape=jax.ShapeDtypeStruct((npad, d), jnp.float32),
        compiler_params=pltpu.CompilerParams(
            dimension_semantics=("arbitrary", "arbitrary")),
    )(meta, x_sorted, W1, b1.reshape(e, 1, h_dim), W2,
      b2.reshape(e, 1, h_dim), W3, b3.reshape(e, 1, d), gate_col)

    dr = dest.reshape(n, TOPK)
    y = out_rows[dr[:, 0]] + out_rows[dr[:, 1]]
    return y


# speedup vs baseline: 1.0552x; 1.0552x over previous
import functools

import jax
import jax.numpy as jnp
from jax import lax
from jax.experimental import pallas as pl
from jax.experimental.pallas import tpu as pltpu

TOPK = 2
BLK = 256
HT = 4

_ERF_P = (1.1272895443e+00, -3.7081442500e-01, 1.0510804251e-01,
          -2.1557972806e-02, 3.1188406981e-03, -3.1129532932e-04,
          2.0792815143e-05, -8.8244686580e-07, 2.1433714458e-08,
          -2.2632114575e-10)


def _gelu(h):
    xc = jnp.clip(h * 0.7071067811865476, -4.2, 4.2)
    u = xc * xc
    p = jnp.float32(_ERF_P[-1])
    for c in _ERF_P[-2::-1]:
        p = p * u + jnp.float32(c)
    return h * 0.5 * (1.0 + xc * p)


def _moe_body(m_ref, x_ref, w1_ref, b1_ref, w2_ref, b2_ref, w3_ref, b3_ref,
              gt_ref, out_ref, w1b, w2b, w3b, hb, *, nb, htile):
    g = pl.program_id(0)
    t = pl.program_id(1)

    @pl.when(g < m_ref[0, nb])
    def _():
        @pl.when(m_ref[1, g] == 1)
        def _():
            w2b[:, pl.ds(t * htile, htile)] = w2_ref[0].astype(jnp.bfloat16)
            w3b[pl.ds(t * htile, htile), :] = w3_ref[0].astype(jnp.bfloat16)

            @pl.when(t == 0)
            def _():
                w1b[...] = w1_ref[0].astype(jnp.bfloat16)

        @pl.when(t == 0)
        def _():
            h = jnp.dot(x_ref[...].astype(jnp.bfloat16), w1b[...],
                        preferred_element_type=jnp.float32) + b1_ref[0]
            hb[...] = _gelu(h).astype(jnp.bfloat16)

        h2 = jnp.dot(hb[...], w2b[:, pl.ds(t * htile, htile)],
                     preferred_element_type=jnp.float32) + b2_ref[0]
        part = jnp.dot(h2.astype(jnp.bfloat16), w3b[pl.ds(t * htile, htile), :],
                       preferred_element_type=jnp.float32)

        @pl.when(t == 0)
        def _():
            out_ref[...] = (part + b3_ref[0]) * gt_ref[...]

        @pl.when(t != 0)
        def _():
            out_ref[...] += part * gt_ref[...]


def kernel(z_pred, expert_eligibility, W1, b1, W2, b2, W3, b3):
    n, d = z_pred.shape
    e = W1.shape[0]
    h_dim = W1.shape[2]
    htile = h_dim // HT
    np_ = n * TOPK
    nb = np_ // BLK + e - 1
    npad = nb * BLK

    vals, idx = lax.top_k(expert_eligibility, TOPK)
    gates = jax.nn.softmax(vals, axis=-1)
    e_flat = idx.reshape(-1).astype(jnp.int32)
    g_flat = gates.reshape(-1)

    onehot = (e_flat[:, None] == jnp.arange(e, dtype=jnp.int32)[None, :])
    csum = jnp.cumsum(onehot.astype(jnp.int32), axis=0)
    rank = jnp.take_along_axis(csum, e_flat[:, None], axis=1)[:, 0] - 1
    counts = csum[-1]
    nblk_e = (counts + BLK - 1) // BLK
    cum_blocks = jnp.cumsum(nblk_e)
    total_blocks = cum_blocks[-1]
    poffs = jnp.concatenate([jnp.zeros(1, jnp.int32),
                             jnp.cumsum(nblk_e * BLK)])[:e]
    dest = poffs[e_flat] + rank

    bids = jnp.arange(nb, dtype=jnp.int32)
    block_expert = jnp.searchsorted(
        cum_blocks, jnp.minimum(bids, total_blocks - 1), side="right"
    ).astype(jnp.int32)
    changed = jnp.concatenate([jnp.ones(1, jnp.int32),
                               (jnp.diff(block_expert) != 0).astype(jnp.int32)])
    meta = jnp.stack([jnp.concatenate([block_expert, total_blocks[None]]),
                      jnp.concatenate([changed, jnp.zeros(1, jnp.int32)])])

    tok_src = jnp.zeros(npad, jnp.int32).at[dest].set(
        jnp.arange(np_, dtype=jnp.int32) // TOPK)
    gate_col = jnp.zeros((npad,), jnp.float32).at[dest].set(
        g_flat).reshape(npad, 1)
    x_sorted = z_pred[tok_src]

    def w2_map(g, t, m):
        return (m[0, g], 0, jnp.where(m[1, g] == 1, t, HT - 1))

    def w3_map(g, t, m):
        return (m[0, g], jnp.where(m[1, g] == 1, t, HT - 1), 0)

    grid_spec = pltpu.PrefetchScalarGridSpec(
        num_scalar_prefetch=1,
        grid=(nb, HT),
        in_specs=[
            pl.BlockSpec((BLK, d), lambda g, t, m: (g, 0)),
            pl.BlockSpec((1, d, h_dim), lambda g, t, m: (m[0, g], 0, 0)),
            pl.BlockSpec((1, 1, h_dim), lambda g, t, m: (m[0, g], 0, 0)),
            pl.BlockSpec((1, h_dim, htile), w2_map),
            pl.BlockSpec((1, 1, htile), lambda g, t, m: (m[0, g], 0, t)),
            pl.BlockSpec((1, htile, d), w3_map),
            pl.BlockSpec((1, 1, d), lambda g, t, m: (m[0, g], 0, 0)),
            pl.BlockSpec((BLK, 1), lambda g, t, m: (g, 0)),
        ],
        out_specs=pl.BlockSpec((BLK, d), lambda g, t, m: (g, 0)),
        scratch_shapes=[
            pltpu.VMEM((d, h_dim), jnp.bfloat16),
            pltpu.VMEM((h_dim, h_dim), jnp.bfloat16),
            pltpu.VMEM((h_dim, d), jnp.bfloat16),
            pltpu.VMEM((BLK, h_dim), jnp.bfloat16),
        ],
    )
    out_rows = pl.pallas_call(
        functools.partial(_moe_body, nb=nb, htile=htile),
        grid_spec=grid_spec,
        out_shape=jax.ShapeDtypeStruct((npad, d), jnp.float32),
        compiler_params=pltpu.CompilerParams(
            dimension_semantics=("arbitrary", "arbitrary")),
    )(meta, x_sorted, W1, b1.reshape(e, 1, h_dim), W2,
      b2.reshape(e, 1, h_dim), W3, b3.reshape(e, 1, d), gate_col)

    dr = dest.reshape(n, TOPK)
    y = out_rows[dr[:, 0]] + out_rows[dr[:, 1]]
    return y


# speedup vs baseline: 1.1499x; 1.0898x over previous
import functools

import jax
import jax.numpy as jnp
from jax import lax
from jax.experimental import pallas as pl
from jax.experimental.pallas import tpu as pltpu

TOPK = 2
BLK = 256
HT = 4


def _moe_body(m_ref, x_ref, w1_ref, b1_ref, w2_ref, b2_ref, w3_ref, b3_ref,
              gt_ref, out_ref, w1b, w2b, w3b, hb, *, nb, htile):
    g = pl.program_id(0)
    t = pl.program_id(1)

    @pl.when(g < m_ref[0, nb])
    def _():
        @pl.when(m_ref[1, g] == 1)
        def _():
            w2b[:, pl.ds(t * htile, htile)] = w2_ref[0].astype(jnp.bfloat16)
            w3b[pl.ds(t * htile, htile), :] = w3_ref[0].astype(jnp.bfloat16)

            @pl.when(t == 0)
            def _():
                w1b[...] = w1_ref[0].astype(jnp.bfloat16)

        @pl.when(t == 0)
        def _():
            h = jnp.dot(x_ref[...].astype(jnp.bfloat16), w1b[...],
                        preferred_element_type=jnp.float32) + b1_ref[0]
            h = h * 0.5 * (1.0 + lax.erf(h * 0.7071067811865476))
            hb[...] = h.astype(jnp.bfloat16)

        h2 = jnp.dot(hb[...], w2b[:, pl.ds(t * htile, htile)],
                     preferred_element_type=jnp.float32) + b2_ref[0]
        part = jnp.dot(h2.astype(jnp.bfloat16), w3b[pl.ds(t * htile, htile), :],
                       preferred_element_type=jnp.float32)

        @pl.when(t == 0)
        def _():
            out_ref[...] = (part + b3_ref[0]) * gt_ref[...]

        @pl.when(t != 0)
        def _():
            out_ref[...] += part * gt_ref[...]


def kernel(z_pred, expert_eligibility, W1, b1, W2, b2, W3, b3):
    n, d = z_pred.shape
    e = W1.shape[0]
    h_dim = W1.shape[2]
    htile = h_dim // HT
    np_ = n * TOPK
    nb = np_ // BLK + e - 1
    npad = nb * BLK

    vals, idx = lax.top_k(expert_eligibility, TOPK)
    gates = jax.nn.softmax(vals, axis=-1)
    e_flat = idx.reshape(-1).astype(jnp.int32)
    g_flat = gates.reshape(-1)

    onehot = (e_flat[:, None] == jnp.arange(e, dtype=jnp.int32)[None, :])
    csum = jnp.cumsum(onehot.astype(jnp.int32), axis=0)
    rank = jnp.take_along_axis(csum, e_flat[:, None], axis=1)[:, 0] - 1
    counts = csum[-1]
    nblk_e = (counts + BLK - 1) // BLK
    cum_blocks = jnp.cumsum(nblk_e)
    total_blocks = cum_blocks[-1]
    poffs = jnp.concatenate([jnp.zeros(1, jnp.int32),
                             jnp.cumsum(nblk_e * BLK)])[:e]
    dest = poffs[e_flat] + rank

    bids = jnp.arange(nb, dtype=jnp.int32)
    block_expert = jnp.searchsorted(
        cum_blocks, jnp.minimum(bids, total_blocks - 1), side="right"
    ).astype(jnp.int32)
    changed = jnp.concatenate([jnp.ones(1, jnp.int32),
                               (jnp.diff(block_expert) != 0).astype(jnp.int32)])
    meta = jnp.stack([jnp.concatenate([block_expert, total_blocks[None]]),
                      jnp.concatenate([changed, jnp.zeros(1, jnp.int32)])])

    tok_src = jnp.zeros(npad, jnp.int32).at[dest].set(
        jnp.arange(np_, dtype=jnp.int32) // TOPK)
    gate_col = jnp.zeros((npad,), jnp.float32).at[dest].set(
        g_flat).reshape(npad, 1)
    x_sorted = z_pred[tok_src]

    def w2_map(g, t, m):
        return (m[0, g], 0, jnp.where(m[1, g] == 1, t, HT - 1))

    def w3_map(g, t, m):
        return (m[0, g], jnp.where(m[1, g] == 1, t, HT - 1), 0)

    grid_spec = pltpu.PrefetchScalarGridSpec(
        num_scalar_prefetch=1,
        grid=(nb, HT),
        in_specs=[
            pl.BlockSpec((BLK, d), lambda g, t, m: (g, 0)),
            pl.BlockSpec((1, d, h_dim), lambda g, t, m: (m[0, g], 0, 0)),
            pl.BlockSpec((1, 1, h_dim), lambda g, t, m: (m[0, g], 0, 0)),
            pl.BlockSpec((1, h_dim, htile), w2_map),
            pl.BlockSpec((1, 1, htile), lambda g, t, m: (m[0, g], 0, t)),
            pl.BlockSpec((1, htile, d), w3_map),
            pl.BlockSpec((1, 1, d), lambda g, t, m: (m[0, g], 0, 0)),
            pl.BlockSpec((BLK, 1), lambda g, t, m: (g, 0)),
        ],
        out_specs=pl.BlockSpec((BLK, d), lambda g, t, m: (g, 0)),
        scratch_shapes=[
            pltpu.VMEM((d, h_dim), jnp.bfloat16),
            pltpu.VMEM((h_dim, h_dim), jnp.bfloat16),
            pltpu.VMEM((h_dim, d), jnp.bfloat16),
            pltpu.VMEM((BLK, h_dim), jnp.bfloat16),
        ],
    )
    out_rows = pl.pallas_call(
        functools.partial(_moe_body, nb=nb, htile=htile),
        grid_spec=grid_spec,
        out_shape=jax.ShapeDtypeStruct((npad, d), jnp.float32),
        compiler_params=pltpu.CompilerParams(
            dimension_semantics=("arbitrary", "arbitrary")),
    )(meta, x_sorted, W1, b1.reshape(e, 1, h_dim), W2,
      b2.reshape(e, 1, h_dim), W3, b3.reshape(e, 1, d), gate_col)

    dr = dest.reshape(n, TOPK)
    y = out_rows[dr[:, 0]] + out_rows[dr[:, 1]]
    return y
